# baseline (speedup 1.0000x reference)
import functools

import jax
import jax.numpy as jnp
from jax import lax
from jax.experimental import pallas as pl
from jax.experimental.pallas import tpu as pltpu
from jax.experimental.pallas import tpu_sc as plsc

_D = 64
_CHUNK = 128
_H = 50
_BT = 128

_INFO = plsc.get_sparse_core_info()
_NC = _INFO.num_cores
_NS = _INFO.num_subcores
_NW = _NC * _NS
_BT_PER_W = _BT // _NW
_N_CHUNKS = _H * _BT_PER_W


def _emb_body(idx_hbm, table_hbm, out_hbm, raw0, raw1, hi0, hi1, par0, par1,
              rows0, rows1, patch0, patch1, rsem, gsem, psem):
    wid = lax.axis_index("s") * _NC + lax.axis_index("c")

    raw = (raw0, raw1)
    hi = (hi0, hi1)
    par = (par0, par1)
    rows = (rows0, rows1)
    patch = (patch0, patch1)

    iota = lax.iota(jnp.int32, 16)
    rot = [(iota + s) % 16 for s in range(16)]

    def idx_slice(j):
        h = j // _BT_PER_W
        k = j % _BT_PER_W
        return idx_hbm.at[h, pl.ds((wid * _BT_PER_W + k) * _CHUNK, _CHUNK)]

    def stage_raw(j, b):
        pltpu.async_copy(idx_slice(j), raw[b], rsem.at[b])

    def wait_raw(j, b):
        pltpu.make_async_copy(idx_slice(j), raw[b], rsem.at[b]).wait()

    def prep_chunk(b):
        for g in range(8):
            v = raw[b][pl.ds(16 * g, 16)]
            hi[b][pl.ds(16 * g, 16)] = lax.shift_right_logical(v, 1)
            par[b][pl.ds(16 * g, 16)] = (v & 1) * 64

    def issue_gather(b):
        pltpu.async_copy(table_hbm.at[hi[b]], rows[b], gsem.at[b])

    def wait_gather(b):
        pltpu.make_async_copy(table_hbm.at[hi[b]], rows[b], gsem.at[b]).wait()

    def transpose_chunk(b):
        def tb_body(tb, carry):
            bl0 = 16 * tb
            bl_vec = iota + bl0
            par_vec = par[b][pl.ds(bl0, 16)]
            for td in range(_D // 16):
                for s in range(16):
                    dvec = rot[s] + (16 * td)
                    vals = plsc.load_gather(rows[b], [bl_vec, par_vec + dvec])
                    plsc.store_scatter(patch[b], [dvec, bl_vec], vals)
            return carry

        lax.fori_loop(0, _CHUNK // 16, tb_body, 0)

    def out_tile(j, dt):
        h = j // _BT_PER_W
        bt = wid * _BT_PER_W + (j % _BT_PER_W)
        return out_hbm.at[h, dt, bt]

    def issue_writes(j, b):
        for dt in range(8):
            pltpu.async_copy(
                patch[b].at[pl.ds(8 * dt, 8)], out_tile(j, dt), psem.at[b])

    def wait_writes(j, b):
        for dt in range(8):
            pltpu.make_async_copy(
                patch[b].at[pl.ds(8 * dt, 8)], out_tile(j, dt),
                psem.at[b]).wait()

    stage_raw(0, 0)
    stage_raw(1, 1)
    wait_raw(0, 0)
    prep_chunk(0)
    issue_gather(0)

    def pair(p, carry):
        for s in range(2):
            j = 2 * p + s

            @pl.when(j + 1 < _N_CHUNKS)
            def _():
                wait_raw(j + 1, 1 - s)
                prep_chunk(1 - s)
                issue_gather(1 - s)

            @pl.when(j + 2 < _N_CHUNKS)
            def _():
                stage_raw(j + 2, s)

            wait_gather(s)

            @pl.when(j >= 2)
            def _():
                wait_writes(j - 2, s)

            transpose_chunk(s)
            issue_writes(j, s)
        return carry

    lax.fori_loop(0, _N_CHUNKS // 2, pair, 0)

    wait_writes(_N_CHUNKS - 2, 0)
    wait_writes(_N_CHUNKS - 1, 1)


@jax.jit
def _emb_call(idx, table2):
    mesh = plsc.VectorSubcoreMesh(core_axis_name="c", subcore_axis_name="s")
    run = pl.kernel(
        _emb_body,
        out_type=jax.ShapeDtypeStruct((_H, 8, _BT, 8, _CHUNK), jnp.float32),
        mesh=mesh,
        scratch_types=[
            pltpu.VMEM((_CHUNK,), jnp.int32),
            pltpu.VMEM((_CHUNK,), jnp.int32),
            pltpu.VMEM((_CHUNK,), jnp.int32),
            pltpu.VMEM((_CHUNK,), jnp.int32),
            pltpu.VMEM((_CHUNK,), jnp.int32),
            pltpu.VMEM((_CHUNK,), jnp.int32),
            pltpu.VMEM((_CHUNK, 2 * _D), jnp.float32),
            pltpu.VMEM((_CHUNK, 2 * _D), jnp.float32),
            pltpu.VMEM((_D, _CHUNK), jnp.float32),
            pltpu.VMEM((_D, _CHUNK), jnp.float32),
            pltpu.SemaphoreType.DMA((2,)),
            pltpu.SemaphoreType.DMA((2,)),
            pltpu.SemaphoreType.DMA((2,)),
        ],
        compiler_params=pltpu.CompilerParams(
            use_tc_tiling_on_sc=True, needs_layout_passes=False),
    )
    return run(idx, table2)


def kernel(token_ids, weight):
    b, h = token_ids.shape
    idx = token_ids.T.astype(jnp.int32)
    v2 = weight.shape[0] // 2
    table2 = (weight.T.reshape(_D, v2, 2)
              .transpose(1, 2, 0)
              .reshape(v2, 2 * _D))
    out5 = _emb_call(idx, table2)
    return out5.transpose(2, 4, 0, 1, 3).reshape(b, h, _D)


# speedup vs baseline: 1.3899x; 1.3899x over previous
import functools

import jax
import jax.numpy as jnp
from jax import lax
from jax.experimental import pallas as pl
from jax.experimental.pallas import tpu as pltpu
from jax.experimental.pallas import tpu_sc as plsc

_D = 64
_CHUNK = 128
_H = 50
_BT = 128

_INFO = plsc.get_sparse_core_info()
_NC = _INFO.num_cores
_NS = _INFO.num_subcores
_NW = _NC * _NS
_BT_PER_W = _BT // _NW
_N_CHUNKS = _H * _BT_PER_W


def _emb_body(idx_hbm, table_hbm, out_hbm, raw0, raw1, hi0, hi1, par0, par1,
              rows0, rows1, patch0, patch1, rsem, gsem, psem):
    wid = lax.axis_index("s") * _NC + lax.axis_index("c")

    raw = (raw0, raw1)
    hi = (hi0, hi1)
    par = (par0, par1)
    rows = (rows0, rows1)
    patch = (patch0, patch1)

    iota = lax.iota(jnp.int32, 16)
    rot = [(iota + s) % 16 for s in range(16)]

    def idx_slice(j):
        h = j // _BT_PER_W
        k = j % _BT_PER_W
        return idx_hbm.at[h, pl.ds((wid * _BT_PER_W + k) * _CHUNK, _CHUNK)]

    def stage_raw(j, b):
        pltpu.async_copy(idx_slice(j), raw[b], rsem.at[b])

    def wait_raw(j, b):
        pltpu.make_async_copy(idx_slice(j), raw[b], rsem.at[b]).wait()

    def prep_chunk(b):
        for g in range(8):
            v = raw[b][pl.ds(16 * g, 16)]
            hi[b][pl.ds(16 * g, 16)] = lax.shift_right_logical(v, 1)
            par[b][pl.ds(16 * g, 16)] = (v & 1) * 64

    def issue_gather(b):
        pltpu.async_copy(table_hbm.at[hi[b]], rows[b], gsem.at[b])

    def wait_gather(b):
        pltpu.make_async_copy(table_hbm.at[hi[b]], rows[b], gsem.at[b]).wait()

    def transpose_chunk(b):
        def tb_body(tb, carry):
            bl0 = 16 * tb
            bl_vec = iota + bl0
            par_vec = par[b][pl.ds(bl0, 16)]
            steps = [(rot[s] + (16 * td), s)
                     for td in range(_D // 16) for s in range(16)]
            pending = []
            for dvec, _ in steps:
                vals = plsc.load_gather(rows[b], [bl_vec, par_vec + dvec])
                pending.append((dvec, vals))
                if len(pending) == 3:
                    pdvec, pvals = pending.pop(0)
                    plsc.store_scatter(patch[b], [pdvec, bl_vec], pvals)
            for pdvec, pvals in pending:
                plsc.store_scatter(patch[b], [pdvec, bl_vec], pvals)
            return carry

        lax.fori_loop(0, _CHUNK // 16, tb_body, 0)

    def out_tile(j, dt):
        h = j // _BT_PER_W
        bt = wid * _BT_PER_W + (j % _BT_PER_W)
        return out_hbm.at[h, dt, bt]

    def issue_writes(j, b):
        for dt in range(8):
            pltpu.async_copy(
                patch[b].at[pl.ds(8 * dt, 8)], out_tile(j, dt), psem.at[b])

    def wait_writes(j, b):
        for dt in range(8):
            pltpu.make_async_copy(
                patch[b].at[pl.ds(8 * dt, 8)], out_tile(j, dt),
                psem.at[b]).wait()

    stage_raw(0, 0)
    stage_raw(1, 1)
    wait_raw(0, 0)
    prep_chunk(0)
    issue_gather(0)

    def pair(p, carry):
        for s in range(2):
            j = 2 * p + s

            @pl.when(j + 1 < _N_CHUNKS)
            def _():
                wait_raw(j + 1, 1 - s)
                prep_chunk(1 - s)
                issue_gather(1 - s)

            @pl.when(j + 2 < _N_CHUNKS)
            def _():
                stage_raw(j + 2, s)

            wait_gather(s)

            @pl.when(j >= 2)
            def _():
                wait_writes(j - 2, s)

            transpose_chunk(s)
            issue_writes(j, s)
        return carry

    lax.fori_loop(0, _N_CHUNKS // 2, pair, 0)

    wait_writes(_N_CHUNKS - 2, 0)
    wait_writes(_N_CHUNKS - 1, 1)


@jax.jit
def _emb_call(idx, table2):
    mesh = plsc.VectorSubcoreMesh(core_axis_name="c", subcore_axis_name="s")
    run = pl.kernel(
        _emb_body,
        out_type=jax.ShapeDtypeStruct((_H, 8, _BT, 8, _CHUNK), jnp.float32),
        mesh=mesh,
        scratch_types=[
            pltpu.VMEM((_CHUNK,), jnp.int32),
            pltpu.VMEM((_CHUNK,), jnp.int32),
            pltpu.VMEM((_CHUNK,), jnp.int32),
            pltpu.VMEM((_CHUNK,), jnp.int32),
            pltpu.VMEM((_CHUNK,), jnp.int32),
            pltpu.VMEM((_CHUNK,), jnp.int32),
            pltpu.VMEM((_CHUNK, 2 * _D), jnp.float32),
            pltpu.VMEM((_CHUNK, 2 * _D), jnp.float32),
            pltpu.VMEM((_D, _CHUNK), jnp.float32),
            pltpu.VMEM((_D, _CHUNK), jnp.float32),
            pltpu.SemaphoreType.DMA((2,)),
            pltpu.SemaphoreType.DMA((2,)),
            pltpu.SemaphoreType.DMA((2,)),
        ],
        compiler_params=pltpu.CompilerParams(
            use_tc_tiling_on_sc=True, needs_layout_passes=False),
    )
    return run(idx, table2)


def kernel(token_ids, weight):
    b, h = token_ids.shape
    idx = token_ids.T.astype(jnp.int32)
    table2 = weight.reshape(weight.shape[0] // 2, 2 * _D)
    out5 = _emb_call(idx, table2)
    return out5.transpose(2, 4, 0, 1, 3).reshape(b, h, _D)


# speedup vs baseline: 1.5134x; 1.0889x over previous
import functools

import jax
import jax.numpy as jnp
from jax import lax
from jax.experimental import pallas as pl
from jax.experimental.pallas import tpu as pltpu
from jax.experimental.pallas import tpu_sc as plsc

_D = 64
_CHUNK = 128
_H = 50
_BT = 128

_INFO = plsc.get_sparse_core_info()
_NC = _INFO.num_cores
_NS = _INFO.num_subcores
_NW = _NC * _NS
_BT_PER_W = _BT // _NW
_N_CHUNKS = _H * _BT_PER_W


def _idx_body(idx_hbm, out_hbm, sem):
    wid = lax.axis_index("s") * _NC + lax.axis_index("c")
    copies = []
    for k in range(_BT_PER_W):
        bt = wid * _BT_PER_W + k
        for hr in range(_H // 8):
            copies.append((idx_hbm.at[pl.ds(8 * hr, 8), pl.ds(bt * _CHUNK, _CHUNK)],
                           out_hbm.at[pl.ds(8 * hr, 8), bt]))
        for h in range(8 * (_H // 8), _H):
            copies.append((idx_hbm.at[h, pl.ds(bt * _CHUNK, _CHUNK)],
                           out_hbm.at[h, bt]))
    for src, dst in copies:
        pltpu.async_copy(src, dst, sem)
    for src, dst in copies:
        pltpu.make_async_copy(src, dst, sem).wait()


@jax.jit
def _idx_call(idx):
    mesh = plsc.VectorSubcoreMesh(core_axis_name="c", subcore_axis_name="s")
    run = pl.kernel(
        _idx_body,
        out_type=jax.ShapeDtypeStruct((_H, _BT, _CHUNK), jnp.int32),
        mesh=mesh,
        scratch_types=[pltpu.SemaphoreType.DMA],
        compiler_params=pltpu.CompilerParams(
            use_tc_tiling_on_sc=True, needs_layout_passes=False),
    )
    return run(idx)


def _emb_body(idx_hbm, table_hbm, out_hbm, raw0, raw1, rows0, rows1, patch0,
              patch1, rsem, gsem, psem):
    wid = lax.axis_index("s") * _NC + lax.axis_index("c")

    raw = (raw0, raw1)
    rows = (rows0, rows1)
    patch = (patch0, patch1)

    iota = lax.iota(jnp.int32, 16)
    rot = [(iota + s) % 16 for s in range(16)]

    def idx_slice(j):
        h = j // _BT_PER_W
        bt = wid * _BT_PER_W + (j % _BT_PER_W)
        return idx_hbm.at[h, bt]

    def stage_raw(j, b):
        pltpu.async_copy(idx_slice(j), raw[b], rsem.at[b])

    def wait_raw(j, b):
        pltpu.make_async_copy(idx_slice(j), raw[b], rsem.at[b]).wait()

    def issue_gather(b):
        pltpu.async_copy(table_hbm.at[raw[b]], rows[b], gsem.at[b])

    def wait_gather(b):
        pltpu.make_async_copy(table_hbm.at[raw[b]], rows[b], gsem.at[b]).wait()

    def transpose_chunk(b):
        def tb_body(tb, carry):
            bl_vec = iota + 16 * tb
            steps = [rot[s] + (16 * td)
                     for td in range(_D // 16) for s in range(16)]
            pending = []
            for dvec in steps:
                vals = plsc.load_gather(rows[b], [bl_vec, dvec])
                pending.append((dvec, vals))
                if len(pending) == 3:
                    pdvec, pvals = pending.pop(0)
                    plsc.store_scatter(patch[b], [pdvec, bl_vec], pvals)
            for pdvec, pvals in pending:
                plsc.store_scatter(patch[b], [pdvec, bl_vec], pvals)
            return carry

        lax.fori_loop(0, _CHUNK // 16, tb_body, 0)

    def out_tile(j, dt):
        h = j // _BT_PER_W
        bt = wid * _BT_PER_W + (j % _BT_PER_W)
        return out_hbm.at[h, dt, bt]

    def issue_writes(j, b):
        for dt in range(8):
            pltpu.async_copy(
                patch[b].at[pl.ds(8 * dt, 8)], out_tile(j, dt), psem.at[b])

    def wait_writes(j, b):
        for dt in range(8):
            pltpu.make_async_copy(
                patch[b].at[pl.ds(8 * dt, 8)], out_tile(j, dt),
                psem.at[b]).wait()

    stage_raw(0, 0)
    stage_raw(1, 1)
    wait_raw(0, 0)
    issue_gather(0)

    def pair(p, carry):
        for s in range(2):
            j = 2 * p + s

            @pl.when(j + 1 < _N_CHUNKS)
            def _():
                wait_raw(j + 1, 1 - s)
                issue_gather(1 - s)

            wait_gather(s)

            @pl.when(j + 2 < _N_CHUNKS)
            def _():
                stage_raw(j + 2, s)

            @pl.when(j >= 2)
            def _():
                wait_writes(j - 2, s)

            transpose_chunk(s)
            issue_writes(j, s)
        return carry

    lax.fori_loop(0, _N_CHUNKS // 2, pair, 0)

    wait_writes(_N_CHUNKS - 2, 0)
    wait_writes(_N_CHUNKS - 1, 1)


@jax.jit
def _emb_call(idx3, weight):
    mesh = plsc.VectorSubcoreMesh(core_axis_name="c", subcore_axis_name="s")
    run = pl.kernel(
        _emb_body,
        out_type=jax.ShapeDtypeStruct((_H, 8, _BT, 8, _CHUNK), jnp.float32),
        mesh=mesh,
        scratch_types=[
            pltpu.VMEM((_CHUNK,), jnp.int32),
            pltpu.VMEM((_CHUNK,), jnp.int32),
            pltpu.VMEM((_CHUNK, _D), jnp.float32),
            pltpu.VMEM((_CHUNK, _D), jnp.float32),
            pltpu.VMEM((_D, _CHUNK), jnp.float32),
            pltpu.VMEM((_D, _CHUNK), jnp.float32),
            pltpu.SemaphoreType.DMA((2,)),
            pltpu.SemaphoreType.DMA((2,)),
            pltpu.SemaphoreType.DMA((2,)),
        ],
        compiler_params=pltpu.CompilerParams(
            use_tc_tiling_on_sc=False, needs_layout_passes=False),
    )
    return run(idx3, weight)


def kernel(token_ids, weight):
    b, h = token_ids.shape
    idx3 = _idx_call(token_ids.T.astype(jnp.int32))
    out5 = _emb_call(idx3, weight)
    return out5.transpose(2, 4, 0, 1, 3).reshape(b, h, _D)


# speedup vs baseline: 1.5416x; 1.0186x over previous
import functools

import jax
import jax.numpy as jnp
from jax import lax
from jax.experimental import pallas as pl
from jax.experimental.pallas import tpu as pltpu
from jax.experimental.pallas import tpu_sc as plsc

_D = 64
_CHUNK = 128
_H = 50
_BT = 128

_INFO = plsc.get_sparse_core_info()
_NC = _INFO.num_cores
_NS = _INFO.num_subcores
_NW = _NC * _NS
_BT_PER_W = _BT // _NW
_N_CHUNKS = _H * _BT_PER_W


def _emb_body(idx_hbm, table_hbm, out_hbm, raw0, raw1, rows0, rows1, patch0,
              patch1, rsem, gsem, psem):
    wid = lax.axis_index("s") * _NC + lax.axis_index("c")

    raw = (raw0, raw1)
    rows = (rows0, rows1)
    patch = (patch0, patch1)

    iota = lax.iota(jnp.int32, 16)
    rot = [(iota + s) % 16 for s in range(16)]

    def idx_slice(j):
        h = j // _BT_PER_W
        k = j % _BT_PER_W
        return idx_hbm.at[h, pl.ds((wid * _BT_PER_W + k) * _CHUNK, _CHUNK)]

    def stage_raw(j, b):
        pltpu.async_copy(idx_slice(j), raw[b], rsem.at[b])

    def wait_raw(j, b):
        pltpu.make_async_copy(idx_slice(j), raw[b], rsem.at[b]).wait()

    def issue_gather(b):
        pltpu.async_copy(table_hbm.at[raw[b]], rows[b], gsem.at[b])

    def wait_gather(b):
        pltpu.make_async_copy(table_hbm.at[raw[b]], rows[b], gsem.at[b]).wait()

    def transpose_chunk(b):
        def tb_body(tb, carry):
            bl_vec = iota + 16 * tb
            steps = [rot[s] + (16 * td)
                     for td in range(_D // 16) for s in range(16)]
            pending = []
            for dvec in steps:
                vals = plsc.load_gather(rows[b], [bl_vec, dvec])
                pending.append((dvec, vals))
                if len(pending) == 3:
                    pdvec, pvals = pending.pop(0)
                    plsc.store_scatter(patch[b], [pdvec, bl_vec], pvals)
            for pdvec, pvals in pending:
                plsc.store_scatter(patch[b], [pdvec, bl_vec], pvals)
            return carry

        lax.fori_loop(0, _CHUNK // 16, tb_body, 0)

    def out_tile(j, dt):
        h = j // _BT_PER_W
        bt = wid * _BT_PER_W + (j % _BT_PER_W)
        return out_hbm.at[h, dt, bt]

    def issue_writes(j, b):
        for dt in range(8):
            pltpu.async_copy(
                patch[b].at[pl.ds(8 * dt, 8)], out_tile(j, dt), psem.at[b])

    def wait_writes(j, b):
        for dt in range(8):
            pltpu.make_async_copy(
                patch[b].at[pl.ds(8 * dt, 8)], out_tile(j, dt),
                psem.at[b]).wait()

    stage_raw(0, 0)
    stage_raw(1, 1)
    wait_raw(0, 0)
    issue_gather(0)

    def pair(p, carry):
        for s in range(2):
            j = 2 * p + s

            @pl.when(j + 1 < _N_CHUNKS)
            def _():
                wait_raw(j + 1, 1 - s)
                issue_gather(1 - s)

            wait_gather(s)

            @pl.when(j + 2 < _N_CHUNKS)
            def _():
                stage_raw(j + 2, s)

            @pl.when(j >= 2)
            def _():
                wait_writes(j - 2, s)

            transpose_chunk(s)
            issue_writes(j, s)
        return carry

    lax.fori_loop(0, _N_CHUNKS // 2, pair, 0)

    wait_writes(_N_CHUNKS - 2, 0)
    wait_writes(_N_CHUNKS - 1, 1)


@jax.jit
def _emb_call(idx, table_p):
    mesh = plsc.VectorSubcoreMesh(core_axis_name="c", subcore_axis_name="s")
    run = pl.kernel(
        _emb_body,
        out_type=jax.ShapeDtypeStruct((_H, 8, _BT, 8, _CHUNK), jnp.float32),
        mesh=mesh,
        scratch_types=[
            pltpu.VMEM((_CHUNK,), jnp.int32),
            pltpu.VMEM((_CHUNK,), jnp.int32),
            pltpu.VMEM((_CHUNK, 2 * _D), jnp.float32),
            pltpu.VMEM((_CHUNK, 2 * _D), jnp.float32),
            pltpu.VMEM((_D, _CHUNK), jnp.float32),
            pltpu.VMEM((_D, _CHUNK), jnp.float32),
            pltpu.SemaphoreType.DMA((2,)),
            pltpu.SemaphoreType.DMA((2,)),
            pltpu.SemaphoreType.DMA((2,)),
        ],
        compiler_params=pltpu.CompilerParams(
            use_tc_tiling_on_sc=True, needs_layout_passes=False),
    )
    return run(idx, table_p)


def kernel(token_ids, weight):
    b, h = token_ids.shape
    idx = token_ids.T.astype(jnp.int32)
    table_p = jnp.pad(weight, ((0, 0), (0, _D)))
    out5 = _emb_call(idx, table_p)
    return out5.transpose(2, 4, 0, 1, 3).reshape(b, h, _D)
